# baseline (speedup 1.0000x reference)
import functools

import jax
import jax.numpy as jnp
from jax import lax
from jax.experimental import pallas as pl
from jax.experimental.pallas import tpu as pltpu
from jax.experimental.pallas import tpu_sc as plsc

_DIM = 96
_GRID = 64
_D = 3 * _DIM
_N = 32768
_TB = 128
_RH = _D // 2


def _build_tables(freq2):
    def body(freq_ref, cs_ref):
        sm = lax.broadcasted_iota(jnp.int32, (24, 128), 1) & 63
        ang = sm.astype(jnp.float32) * freq_ref[...]
        cbits = lax.bitcast_convert_type(
            jnp.cos(ang).astype(jnp.bfloat16), jnp.uint16).astype(jnp.uint32)
        sbits = lax.bitcast_convert_type(
            jnp.sin(ang).astype(jnp.bfloat16), jnp.uint16).astype(jnp.uint32)
        cs_ref[...] = ((cbits << 16) | sbits).astype(jnp.int32)

    return pl.pallas_call(
        body,
        out_shape=jax.ShapeDtypeStruct((24, 128), jnp.int32),
    )(freq2)


def _rope_sc(featT, idxT, cs_flat):
    info = plsc.get_sparse_core_info()
    nc = info.num_cores
    nw = nc * info.num_subcores
    per_w = _N // nw
    nblk = per_w // _TB
    nch = nblk * 2
    mesh = plsc.VectorSubcoreMesh(core_axis_name="c", subcore_axis_name="s")

    @functools.partial(
        pl.kernel,
        mesh=mesh,
        compiler_params=pltpu.CompilerParams(needs_layout_passes=False),
        out_type=jax.ShapeDtypeStruct((_D, _N), jnp.float32),
        scratch_types=[
            pltpu.VMEM((_RH, _TB), jnp.float32),
            pltpu.VMEM((_RH, _TB), jnp.float32),
            pltpu.VMEM((_RH, _TB), jnp.float32),
            pltpu.VMEM((_RH, _TB), jnp.float32),
            pltpu.VMEM((4, _N // 32), jnp.int32),
            pltpu.VMEM((48 * _GRID,), jnp.int32),
            pltpu.SemaphoreType.DMA,
            pltpu.SemaphoreType.DMA,
            pltpu.SemaphoreType.DMA,
            pltpu.SemaphoreType.DMA,
        ],
    )
    def k(feat_hbm, idx_hbm, cs_hbm, out_hbm,
          fbuf0, fbuf1, obuf0, obuf1, cbuf, csb,
          sem_i0, sem_i1, sem_o0, sem_o1):
        wid = lax.axis_index("s") * nc + lax.axis_index("c")
        tok0 = pl.multiple_of(wid * per_w, _TB)
        pltpu.sync_copy(cs_hbm, csb)
        pltpu.sync_copy(idx_hbm.at[:, pl.ds(tok0, per_w)], cbuf)
        fbufs, obufs = (fbuf0, fbuf1), (obuf0, obuf1)
        sems_i, sems_o = (sem_i0, sem_i1), (sem_o0, sem_o1)

        def feat_win(b, h):
            base = pl.multiple_of(tok0 + b * _TB, _TB)
            return feat_hbm.at[pl.ds(h * _RH, _RH), pl.ds(base, _TB)]

        def out_win(b, h):
            base = pl.multiple_of(tok0 + b * _TB, _TB)
            return out_hbm.at[pl.ds(h * _RH, _RH), pl.ds(base, _TB)]

        def compute(b, h, fbuf, obuf):
            @plsc.parallel_loop(0, _TB // 16)
            def grp(g):
                col = b * _TB + g * 16
                axes = (0, 1) if h == 0 else (1, 2)
                sv = {a: cbuf[3 - a, pl.ds(col, 16)] for a in axes}
                c0 = g * 16
                nb = 12
                for lr0 in range(0, _RH, 2 * nb):
                    pairs = []
                    for q in range(nb):
                        lr = lr0 + 2 * q
                        gr = h * _RH + lr
                        a = gr // _DIM
                        kk = (gr % _DIM) // 2
                        pairs.append((lr, a, kk))
                    f0s = [fbuf[lr, pl.ds(c0, 16)] for lr, _, _ in pairs]
                    f1s = [fbuf[lr + 1, pl.ds(c0, 16)] for lr, _, _ in pairs]
                    css = [plsc.load_gather(
                               csb.at[pl.ds(kk * _GRID, _GRID)], [sv[a]])
                           for _, a, kk in pairs]
                    for q, (lr, _, _) in enumerate(pairs):
                        cv = plsc.bitcast(css[q] & jnp.int32(-65536),
                                          jnp.float32)
                        sn = plsc.bitcast(css[q] << 16, jnp.float32)
                        obuf[lr, pl.ds(c0, 16)] = (
                            f0s[q] * cv - f1s[q] * sn)
                        obuf[lr + 1, pl.ds(c0, 16)] = (
                            f1s[q] * cv + f0s[q] * sn)

        pltpu.async_copy(feat_win(0, 0), fbuf0, sem_i0)
        pltpu.async_copy(feat_win(0, 1), fbuf1, sem_i1)

        def pair(p, carry):
            for h in range(2):
                c = p * 2 + h
                pltpu.make_async_copy(feat_win(0, 0), fbufs[h], sems_i[h]).wait()

                @pl.when(p > 0)
                def _():
                    pltpu.make_async_copy(
                        obufs[h], out_win(0, 0), sems_o[h]).wait()

                compute(p, h, fbufs[h], obufs[h])
                pltpu.async_copy(obufs[h], out_win(p, h), sems_o[h])

                @pl.when(c + 2 < nch)
                def _():
                    pltpu.async_copy(feat_win(p + 1, h), fbufs[h], sems_i[h])
            return carry

        lax.fori_loop(0, nblk, pair, 0)
        pltpu.make_async_copy(obuf0, out_win(0, 0), sem_o0).wait()
        pltpu.make_async_copy(obuf1, out_win(0, 1), sem_o1).wait()

    return k(featT, idxT, cs_flat)


def kernel(features, indices, freq):
    freq2 = jnp.repeat(freq.astype(jnp.float32), _GRID).reshape(24, 128)
    cs_t = _build_tables(freq2)
    outT = _rope_sc(features.T, indices.astype(jnp.int32).T,
                    cs_t.reshape(-1))
    return outT.T


# speedup vs baseline: 1.0093x; 1.0093x over previous
import functools

import jax
import jax.numpy as jnp
from jax import lax
from jax.experimental import pallas as pl
from jax.experimental.pallas import tpu as pltpu
from jax.experimental.pallas import tpu_sc as plsc

_DIM = 96
_GRID = 64
_D = 3 * _DIM
_N = 32768
_TB = 128
_RH = _D // 2


def _build_tables(freq2):
    def body(freq_ref, cs_ref):
        sm = lax.broadcasted_iota(jnp.int32, (24, 128), 1) & 63
        ang = sm.astype(jnp.float32) * freq_ref[...]
        cbits = lax.bitcast_convert_type(
            jnp.cos(ang).astype(jnp.bfloat16), jnp.uint16).astype(jnp.uint32)
        sbits = lax.bitcast_convert_type(
            jnp.sin(ang).astype(jnp.bfloat16), jnp.uint16).astype(jnp.uint32)
        cs_ref[...] = ((cbits << 16) | sbits).astype(jnp.int32)

    return pl.pallas_call(
        body,
        out_shape=jax.ShapeDtypeStruct((24, 128), jnp.int32),
    )(freq2)


def _rope_sc(featT, idxT, cs_flat):
    info = plsc.get_sparse_core_info()
    nc = info.num_cores
    nw = nc * info.num_subcores
    per_w = _N // nw
    nblk = per_w // _TB
    nch = nblk * 2
    mesh = plsc.VectorSubcoreMesh(core_axis_name="c", subcore_axis_name="s")

    @functools.partial(
        pl.kernel,
        mesh=mesh,
        compiler_params=pltpu.CompilerParams(needs_layout_passes=False),
        out_type=jax.ShapeDtypeStruct((_D, _N), jnp.float32),
        scratch_types=[
            pltpu.VMEM((_RH, _TB), jnp.float32),
            pltpu.VMEM((_RH, _TB), jnp.float32),
            pltpu.VMEM((_RH, _TB), jnp.float32),
            pltpu.VMEM((_RH, _TB), jnp.float32),
            pltpu.VMEM((4, _N // 32), jnp.int32),
            pltpu.VMEM((48 * _GRID,), jnp.int32),
            pltpu.SemaphoreType.DMA,
            pltpu.SemaphoreType.DMA,
            pltpu.SemaphoreType.DMA,
            pltpu.SemaphoreType.DMA,
        ],
    )
    def k(feat_hbm, idx_hbm, cs_hbm, out_hbm,
          fbuf0, fbuf1, obuf0, obuf1, cbuf, csb,
          sem_i0, sem_i1, sem_o0, sem_o1):
        wid = lax.axis_index("s") * nc + lax.axis_index("c")
        tok0 = pl.multiple_of(wid * per_w, _TB)
        fbufs, obufs = (fbuf0, fbuf1), (obuf0, obuf1)
        sems_i, sems_o = (sem_i0, sem_i1), (sem_o0, sem_o1)

        def feat_win(b, h):
            base = pl.multiple_of(tok0 + b * _TB, _TB)
            return feat_hbm.at[pl.ds(h * _RH, _RH), pl.ds(base, _TB)]

        def out_win(b, h):
            base = pl.multiple_of(tok0 + b * _TB, _TB)
            return out_hbm.at[pl.ds(h * _RH, _RH), pl.ds(base, _TB)]

        def compute(b, h, fbuf, obuf):
            @plsc.parallel_loop(0, _TB // 16)
            def grp(g):
                col = b * _TB + g * 16
                axes = (0, 1) if h == 0 else (1, 2)
                sv = {a: cbuf[3 - a, pl.ds(col, 16)] for a in axes}
                c0 = g * 16
                nb = 12
                for lr0 in range(0, _RH, 2 * nb):
                    pairs = []
                    for q in range(nb):
                        lr = lr0 + 2 * q
                        gr = h * _RH + lr
                        a = gr // _DIM
                        kk = (gr % _DIM) // 2
                        pairs.append((lr, a, kk))
                    f0s = [fbuf[lr, pl.ds(c0, 16)] for lr, _, _ in pairs]
                    f1s = [fbuf[lr + 1, pl.ds(c0, 16)] for lr, _, _ in pairs]
                    css = [plsc.load_gather(
                               csb.at[pl.ds(kk * _GRID, _GRID)], [sv[a]])
                           for _, a, kk in pairs]
                    for q, (lr, _, _) in enumerate(pairs):
                        cv = plsc.bitcast(css[q] & jnp.int32(-65536),
                                          jnp.float32)
                        sn = plsc.bitcast(css[q] << 16, jnp.float32)
                        obuf[lr, pl.ds(c0, 16)] = (
                            f0s[q] * cv - f1s[q] * sn)
                        obuf[lr + 1, pl.ds(c0, 16)] = (
                            f1s[q] * cv + f0s[q] * sn)

        pltpu.async_copy(feat_win(0, 0), fbuf0, sem_i0)
        pltpu.async_copy(feat_win(0, 1), fbuf1, sem_i1)
        pltpu.sync_copy(cs_hbm, csb)
        pltpu.sync_copy(idx_hbm.at[:, pl.ds(tok0, per_w)], cbuf)

        def pair(p, carry):
            for h in range(2):
                c = p * 2 + h
                pltpu.make_async_copy(feat_win(0, 0), fbufs[h], sems_i[h]).wait()

                @pl.when(p > 0)
                def _():
                    pltpu.make_async_copy(
                        obufs[h], out_win(0, 0), sems_o[h]).wait()

                compute(p, h, fbufs[h], obufs[h])
                pltpu.async_copy(obufs[h], out_win(p, h), sems_o[h])

                @pl.when(c + 2 < nch)
                def _():
                    pltpu.async_copy(feat_win(p + 1, h), fbufs[h], sems_i[h])
            return carry

        lax.fori_loop(0, nblk, pair, 0)
        pltpu.make_async_copy(obuf0, out_win(0, 0), sem_o0).wait()
        pltpu.make_async_copy(obuf1, out_win(0, 1), sem_o1).wait()

    return k(featT, idxT, cs_flat)


def kernel(features, indices, freq):
    freq2 = jnp.repeat(freq.astype(jnp.float32), _GRID).reshape(24, 128)
    cs_t = _build_tables(freq2)
    outT = _rope_sc(features.T, indices.astype(jnp.int32).T,
                    cs_t.reshape(-1))
    return outT.T
